# baseline (speedup 1.0000x reference)
import jax
import jax.numpy as jnp
from jax import lax
from jax.experimental import pallas as pl
from jax.experimental.pallas import tpu as pltpu

B = 1024
C = 3
S = 12
P = 32
D = P * P

NXC = 8
BCH = B // NXC


def _body(x_hbm, idx_smem, table_hbm, out_hbm, rows_vmem, sem_x, sem_g, sem_r):
    for k in range(NXC):
        pltpu.make_async_copy(
            x_hbm.at[pl.ds(k * BCH, BCH)],
            out_hbm.at[pl.ds(k * BCH, BCH), pl.ds(0, C)],
            sem_x,
        ).start()

    def issue(i, carry):
        r = idx_smem[i]
        pltpu.make_async_copy(
            table_hbm.at[pl.ds(r, 1)], rows_vmem.at[pl.ds(i, 1)], sem_g
        ).start()
        return carry

    lax.fori_loop(0, B, issue, 0)

    def drain(i, carry):
        pltpu.make_async_copy(
            table_hbm.at[pl.ds(0, 1)], rows_vmem.at[pl.ds(i, 1)], sem_g
        ).wait()
        return carry

    lax.fori_loop(0, B, drain, 0)

    for s in range(S):
        pltpu.make_async_copy(rows_vmem, out_hbm.at[:, C, s], sem_r).start()
    for s in range(S):
        pltpu.make_async_copy(rows_vmem, out_hbm.at[:, C, s], sem_r).wait()
    for k in range(NXC):
        pltpu.make_async_copy(
            x_hbm.at[pl.ds(k * BCH, BCH)],
            out_hbm.at[pl.ds(k * BCH, BCH), pl.ds(0, C)],
            sem_x,
        ).wait()


def kernel(x, id, table):
    x4 = x.reshape(B, C, S, D)
    out4 = pl.pallas_call(
        _body,
        out_shape=jax.ShapeDtypeStruct((B, C + 1, S, D), jnp.float32),
        in_specs=[
            pl.BlockSpec(memory_space=pl.MemorySpace.ANY),
            pl.BlockSpec(memory_space=pltpu.SMEM),
            pl.BlockSpec(memory_space=pl.MemorySpace.ANY),
        ],
        out_specs=pl.BlockSpec(memory_space=pl.MemorySpace.ANY),
        scratch_shapes=[
            pltpu.VMEM((B, D), jnp.float32),
            pltpu.SemaphoreType.DMA,
            pltpu.SemaphoreType.DMA,
            pltpu.SemaphoreType.DMA,
        ],
    )(x4, id, table)
    return out4.reshape(B, C + 1, S, P, P)


# speedup vs baseline: 11.6450x; 11.6450x over previous
import jax
import jax.numpy as jnp
from jax import lax
from jax.experimental import pallas as pl
from jax.experimental.pallas import tpu as pltpu

B = 1024
C = 3
S = 12
P = 32
D = P * P

GB = 32
NCH = B // GB
NBUF = 2


def _body(x_hbm, idx_smem, table_hbm, out_hbm, xbuf, rows_vmem,
          sem_g, sem_r, sem_in, sem_out):
    def copy_in(i):
        return pltpu.make_async_copy(
            x_hbm.at[pl.ds(i * GB, GB)], xbuf.at[i % NBUF], sem_in)

    def copy_out(i):
        return pltpu.make_async_copy(
            xbuf.at[i % NBUF],
            out_hbm.at[pl.ds(i * GB, GB), pl.ds(0, C)], sem_out)

    copy_in(0).start()

    def issue(i, carry):
        r = idx_smem[i]
        pltpu.make_async_copy(
            table_hbm.at[pl.ds(r, 1)], rows_vmem.at[pl.ds(i, 1)], sem_g
        ).start()
        return carry

    lax.fori_loop(0, B, issue, 0)

    def drain(i, carry):
        pltpu.make_async_copy(
            table_hbm.at[pl.ds(0, 1)], rows_vmem.at[pl.ds(i, 1)], sem_g
        ).wait()
        return carry

    lax.fori_loop(0, B, drain, 0)

    for s in range(S):
        pltpu.make_async_copy(rows_vmem, out_hbm.at[:, C, s], sem_r).start()

    for i in range(NCH):
        if i >= 1:
            copy_out(i - 1).wait()
        if i + 1 < NCH:
            copy_in(i + 1).start()
        copy_in(i).wait()
        copy_out(i).start()
    copy_out(NCH - 1).wait()

    for s in range(S):
        pltpu.make_async_copy(rows_vmem, out_hbm.at[:, C, s], sem_r).wait()


def kernel(x, id, table):
    x4 = x.reshape(B, C, S, D)
    out4 = pl.pallas_call(
        _body,
        out_shape=jax.ShapeDtypeStruct((B, C + 1, S, D), jnp.float32),
        in_specs=[
            pl.BlockSpec(memory_space=pl.MemorySpace.ANY),
            pl.BlockSpec(memory_space=pltpu.SMEM),
            pl.BlockSpec(memory_space=pl.MemorySpace.ANY),
        ],
        out_specs=pl.BlockSpec(memory_space=pl.MemorySpace.ANY),
        scratch_shapes=[
            pltpu.VMEM((NBUF, GB, C, S, D), jnp.float32),
            pltpu.VMEM((B, D), jnp.float32),
            pltpu.SemaphoreType.DMA,
            pltpu.SemaphoreType.DMA,
            pltpu.SemaphoreType.DMA,
            pltpu.SemaphoreType.DMA,
        ],
    )(x4, id, table)
    return out4.reshape(B, C + 1, S, P, P)


# speedup vs baseline: 11.9579x; 1.0269x over previous
import jax
import jax.numpy as jnp
from jax import lax
from jax.experimental import pallas as pl
from jax.experimental.pallas import tpu as pltpu

B = 1024
C = 3
S = 12
P = 32
D = P * P

GB = 32
NCH = B // GB
NBUF = 4
GPC = B // NCH


def _body(x_hbm, idx_smem, table_hbm, out_hbm, xbuf, rows_vmem,
          sem_g, sem_r, sem_in, sem_out):
    def copy_in(i):
        return pltpu.make_async_copy(
            x_hbm.at[pl.ds(i * GB, GB)], xbuf.at[i % NBUF], sem_in)

    def copy_out(i):
        return pltpu.make_async_copy(
            xbuf.at[i % NBUF],
            out_hbm.at[pl.ds(i * GB, GB), pl.ds(0, C)], sem_out)

    for b in range(NBUF):
        copy_in(b).start()

    for j in range(NCH):
        for g in range(GPC):
            i = j * GPC + g
            r = idx_smem[i]
            pltpu.make_async_copy(
                table_hbm.at[pl.ds(r, 1)], rows_vmem.at[pl.ds(i, 1)], sem_g
            ).start()
        if j >= 1:
            copy_out(j - 1).wait()
            nxt = j - 1 + NBUF
            if nxt < NCH:
                copy_in(nxt).start()
        copy_in(j).wait()
        copy_out(j).start()

    pltpu.make_async_copy(table_hbm.at[pl.ds(0, B)], rows_vmem, sem_g).wait()

    for s in range(S):
        pltpu.make_async_copy(rows_vmem, out_hbm.at[:, C, s], sem_r).start()

    copy_out(NCH - 1).wait()
    for s in range(S):
        pltpu.make_async_copy(rows_vmem, out_hbm.at[:, C, s], sem_r).wait()


def kernel(x, id, table):
    x4 = x.reshape(B, C, S, D)
    out4 = pl.pallas_call(
        _body,
        out_shape=jax.ShapeDtypeStruct((B, C + 1, S, D), jnp.float32),
        in_specs=[
            pl.BlockSpec(memory_space=pl.MemorySpace.ANY),
            pl.BlockSpec(memory_space=pltpu.SMEM),
            pl.BlockSpec(memory_space=pl.MemorySpace.ANY),
        ],
        out_specs=pl.BlockSpec(memory_space=pl.MemorySpace.ANY),
        scratch_shapes=[
            pltpu.VMEM((NBUF, GB, C, S, D), jnp.float32),
            pltpu.VMEM((B, D), jnp.float32),
            pltpu.SemaphoreType.DMA,
            pltpu.SemaphoreType.DMA,
            pltpu.SemaphoreType.DMA,
            pltpu.SemaphoreType.DMA,
        ],
    )(x4, id, table)
    return out4.reshape(B, C + 1, S, P, P)


# speedup vs baseline: 53.8738x; 4.5053x over previous
import jax
import jax.numpy as jnp
from jax import lax
from jax.experimental import pallas as pl
from jax.experimental.pallas import tpu as pltpu

B = 1024
C = 3
S = 12
P = 32
D = P * P

NSLAB = C * S
NBUF = 4
TB = 128


def _body(x_hbm, idx_smem, table_hbm, out_hbm, xbuf, rows, rows_t,
          sem_g, sem_r, sem_in, sem_out):
    def copy_in(i):
        return pltpu.make_async_copy(
            x_hbm.at[i // S, i % S], xbuf.at[i % NBUF], sem_in)

    def copy_out(i):
        return pltpu.make_async_copy(
            xbuf.at[i % NBUF], out_hbm.at[i // S, i % S], sem_out)

    for b in range(NBUF):
        copy_in(b).start()

    def issue(i, carry):
        r = idx_smem[i]
        pltpu.make_async_copy(
            table_hbm.at[pl.ds(r, 1)], rows.at[pl.ds(i, 1)], sem_g
        ).start()
        return carry

    lax.fori_loop(0, B, issue, 0)

    pltpu.make_async_copy(table_hbm.at[pl.ds(0, B)], rows, sem_g).wait()

    for i in range(B // TB):
        for j in range(D // TB):
            t = rows[pl.ds(i * TB, TB), pl.ds(j * TB, TB)]
            rows_t[pl.ds(j * TB, TB), pl.ds(i * TB, TB)] = t.T

    for s in range(S):
        pltpu.make_async_copy(rows_t, out_hbm.at[C, s], sem_r).start()

    for j in range(NSLAB):
        if j >= 1:
            copy_out(j - 1).wait()
            nxt = j - 1 + NBUF
            if nxt < NSLAB:
                copy_in(nxt).start()
        copy_in(j).wait()
        copy_out(j).start()
    copy_out(NSLAB - 1).wait()

    for s in range(S):
        pltpu.make_async_copy(rows_t, out_hbm.at[C, s], sem_r).wait()


def kernel(x, id, table):
    x_t = jnp.transpose(x.reshape(B, C, S, D), (1, 2, 3, 0))
    out_t = pl.pallas_call(
        _body,
        out_shape=jax.ShapeDtypeStruct((C + 1, S, D, B), jnp.float32),
        in_specs=[
            pl.BlockSpec(memory_space=pl.MemorySpace.ANY),
            pl.BlockSpec(memory_space=pltpu.SMEM),
            pl.BlockSpec(memory_space=pl.MemorySpace.ANY),
        ],
        out_specs=pl.BlockSpec(memory_space=pl.MemorySpace.ANY),
        scratch_shapes=[
            pltpu.VMEM((NBUF, D, B), jnp.float32),
            pltpu.VMEM((B, D), jnp.float32),
            pltpu.VMEM((D, B), jnp.float32),
            pltpu.SemaphoreType.DMA,
            pltpu.SemaphoreType.DMA,
            pltpu.SemaphoreType.DMA,
            pltpu.SemaphoreType.DMA,
        ],
    )(x_t, id, table)
    return jnp.transpose(out_t, (3, 0, 1, 2)).reshape(B, C + 1, S, P, P)


# speedup vs baseline: 57.0204x; 1.0584x over previous
import jax
import jax.numpy as jnp
from jax import lax
from jax.experimental import pallas as pl
from jax.experimental.pallas import tpu as pltpu

B = 1024
C = 3
S = 12
P = 32
D = P * P

NSLAB = C * S
NBUF = 6
TB = 128
GITER = 12
GPJ = -(-B // GITER)
TPOSE_AT = GITER + 1


def _body(x_hbm, idx_smem, table_hbm, out_hbm, xbuf, rows, rows_t,
          sem_g, sem_r, sem_in, sem_out):
    def copy_in(i):
        return pltpu.make_async_copy(
            x_hbm.at[i // S, i % S], xbuf.at[i % NBUF], sem_in)

    def copy_out(i):
        return pltpu.make_async_copy(
            xbuf.at[i % NBUF], out_hbm.at[i // S, i % S], sem_out)

    for b in range(NBUF):
        copy_in(b).start()

    def issue(i, carry):
        r = idx_smem[i]
        pltpu.make_async_copy(
            table_hbm.at[pl.ds(r, 1)], rows.at[pl.ds(i, 1)], sem_g
        ).start()
        return carry

    for j in range(NSLAB):
        if j < GITER:
            lax.fori_loop(j * GPJ, min((j + 1) * GPJ, B), issue, 0)
        if j == TPOSE_AT:
            pltpu.make_async_copy(table_hbm.at[pl.ds(0, B)], rows, sem_g).wait()
            for ib in range(B // TB):
                for jb in range(D // TB):
                    t = rows[pl.ds(ib * TB, TB), pl.ds(jb * TB, TB)]
                    rows_t[pl.ds(jb * TB, TB), pl.ds(ib * TB, TB)] = t.T
            for s in range(S):
                pltpu.make_async_copy(rows_t, out_hbm.at[C, s], sem_r).start()
        if j >= 1:
            copy_out(j - 1).wait()
            nxt = j - 1 + NBUF
            if nxt < NSLAB:
                copy_in(nxt).start()
        copy_in(j).wait()
        copy_out(j).start()
    copy_out(NSLAB - 1).wait()

    for s in range(S):
        pltpu.make_async_copy(rows_t, out_hbm.at[C, s], sem_r).wait()


def kernel(x, id, table):
    x_t = jnp.transpose(x.reshape(B, C, S, D), (1, 2, 3, 0))
    out_t = pl.pallas_call(
        _body,
        out_shape=jax.ShapeDtypeStruct((C + 1, S, D, B), jnp.float32),
        in_specs=[
            pl.BlockSpec(memory_space=pl.MemorySpace.ANY),
            pl.BlockSpec(memory_space=pltpu.SMEM),
            pl.BlockSpec(memory_space=pl.MemorySpace.ANY),
        ],
        out_specs=pl.BlockSpec(memory_space=pl.MemorySpace.ANY),
        scratch_shapes=[
            pltpu.VMEM((NBUF, D, B), jnp.float32),
            pltpu.VMEM((B, D), jnp.float32),
            pltpu.VMEM((D, B), jnp.float32),
            pltpu.SemaphoreType.DMA,
            pltpu.SemaphoreType.DMA,
            pltpu.SemaphoreType.DMA,
            pltpu.SemaphoreType.DMA,
        ],
    )(x_t, id, table)
    return jnp.transpose(out_t, (3, 0, 1, 2)).reshape(B, C + 1, S, P, P)
